# Initial kernel scaffold; baseline (speedup 1.0000x reference)
import jax
import jax.numpy as jnp
from jax.experimental import pallas as pl


def kernel(x, edge_index, params):
    raise NotImplementedError("write your pallas kernel here")


# speedup vs baseline: 5.6176x; 5.6176x over previous
import functools

import jax
import jax.numpy as jnp
from jax import lax
from jax.experimental import pallas as pl
from jax.experimental.pallas import tpu as pltpu
from jax.experimental.pallas import tpu_sc as plsc

_N = 10000
_E = 320000
_D = 128
_NC = 2
_NS = 16
_W = _NC * _NS
_K = 128
_C = _E // _K
_RPT = 632
_NPAD = _RPT * _NS
_BN_EPS = 1e-5


def _segment_sum_sc(h, src_chunks, dst_chunks, zrow):
    mesh = plsc.VectorSubcoreMesh(core_axis_name="c", subcore_axis_name="s")

    @functools.partial(
        pl.kernel,
        out_type=jax.ShapeDtypeStruct((_NC, _NPAD, _D), jnp.float32),
        mesh=mesh,
        scratch_types=[
            pltpu.VMEM((_K,), jnp.int32),
            pltpu.VMEM((_K,), jnp.int32),
            pltpu.VMEM((_K, _D), jnp.float32),
            pltpu.VMEM_SHARED((_NPAD, _D), jnp.float32),
            pltpu.SemaphoreType.DMA,
        ],
    )
    def k(h_hbm, src_hbm, dst_hbm, z_hbm, out_hbm, src_v, dst_v, rows_v,
          acc_sh, sem):
        c = lax.axis_index("c")
        s = lax.axis_index("s")
        wid = s * _NC + c

        pltpu.sync_copy(z_hbm, acc_sh.at[pl.ds(s * _RPT, _RPT)])
        plsc.subcore_barrier()

        n_chunks = (_C - wid + _W - 1) // _W

        def body(i, carry):
            j = wid + i * _W
            pltpu.sync_copy(src_hbm.at[j], src_v)
            pltpu.sync_copy(dst_hbm.at[j], dst_v)
            pltpu.async_copy(h_hbm.at[src_v], rows_v, sem).wait()
            pltpu.sync_copy(rows_v, acc_sh.at[dst_v], add=True)
            return carry

        lax.fori_loop(0, n_chunks, body, 0)
        plsc.subcore_barrier()
        pltpu.sync_copy(acc_sh.at[pl.ds(s * _RPT, _RPT)],
                        out_hbm.at[c, pl.ds(s * _RPT, _RPT)])

    return k(h, src_chunks, dst_chunks, zrow)


def _gin_mlp_body(h_ref, p_ref, sc_ref, w1_ref, b1_ref, mg_ref, mb_ref,
                  w2_ref, b2_ref, g_ref, bb_ref, out_ref):
    pooled = p_ref[0, :_N] + p_ref[1, :_N] + sc_ref[0, 0] * h_ref[...]
    hm = jnp.dot(pooled, w1_ref[...], preferred_element_type=jnp.float32)
    hm = hm + b1_ref[...]
    m = jnp.mean(hm, axis=0, keepdims=True)
    v = jnp.mean((hm - m) ** 2, axis=0, keepdims=True)
    hm = (hm - m) / jnp.sqrt(v + _BN_EPS) * mg_ref[...] + mb_ref[...]
    hm = jnp.maximum(hm, 0.0)
    h2 = jnp.dot(hm, w2_ref[...], preferred_element_type=jnp.float32)
    h2 = h2 + b2_ref[...]
    m2 = jnp.mean(h2, axis=0, keepdims=True)
    v2 = jnp.mean((h2 - m2) ** 2, axis=0, keepdims=True)
    h2 = (h2 - m2) / jnp.sqrt(v2 + _BN_EPS) * g_ref[...] + bb_ref[...]
    out_ref[...] = jnp.maximum(h2, 0.0)


def _gin_mlp_tc(h, parts, scale, w1, b1, mg, mb, w2, b2, g, bb):
    smem = pl.BlockSpec(memory_space=pltpu.SMEM)
    vmem = pl.BlockSpec(memory_space=pltpu.VMEM)
    return pl.pallas_call(
        _gin_mlp_body,
        out_shape=jax.ShapeDtypeStruct((_N, _D), jnp.float32),
        in_specs=[vmem, vmem, smem] + [vmem] * 8,
        out_specs=vmem,
    )(h, parts, scale, w1, b1, mg, mb, w2, b2, g, bb)


def _prod_rows(x):
    n = x.shape[0]
    size = 1
    while size < n:
        size *= 2
    if size > n:
        x = jnp.concatenate([x, jnp.ones((size - n, x.shape[1]), x.dtype)],
                            axis=0)
    while size > 1:
        size //= 2
        x = x[:size] * x[size:]
    return x


def _readout_body(h_ref, cw_ref, cc_ref, cv_ref, cvb_ref, pw_ref, pb_ref,
                  out_ref):
    fea = jnp.dot(h_ref[...], cw_ref[...], preferred_element_type=jnp.float32)
    fea = fea + cc_ref[...]
    p = _prod_rows(fea)
    ro = jnp.dot(p, cv_ref[...], preferred_element_type=jnp.float32)
    ro = ro + cvb_ref[...]
    out_ref[...] = (jnp.dot(ro, pw_ref[...], preferred_element_type=jnp.float32)
                    + pb_ref[...])


def _readout_tc(h, cw, cc, cv, cvb, pw, pb):
    return pl.pallas_call(
        _readout_body,
        out_shape=jax.ShapeDtypeStruct((1, 10), jnp.float32),
    )(h, cw, cc, cv, cvb, pw, pb)


def _readout_args(r):
    cw = r['cpW'][:_D]
    cc = (r['cpW'][_D] + r['cpb']).reshape(1, -1)
    return (cw, cc, r['cpV'], r['cpVb'].reshape(1, -1), r['predW'],
            r['predb'].reshape(1, -1))


def kernel(x, edge_index, params):
    src_chunks = edge_index[0].reshape(_C, _K)
    dst_chunks = edge_index[1].reshape(_C, _K)
    zrow = jnp.zeros((_RPT, _D), jnp.float32)

    h = x
    scores = []
    for l in range(2):
        p = params['gnn'][l]
        scores.append(_readout_tc(h, *_readout_args(params['readout'][l])))
        parts = _segment_sum_sc(h, src_chunks, dst_chunks, zrow)
        scale = (1.0 + params['eps'][l]).reshape(1, 1)
        h = _gin_mlp_tc(
            h, parts, scale,
            p['W1'], p['b1'].reshape(1, -1),
            p['mbn_g'].reshape(1, -1), p['mbn_b'].reshape(1, -1),
            p['W2'], p['b2'].reshape(1, -1),
            p['bn_g'].reshape(1, -1), p['bn_b'].reshape(1, -1))
    scores.append(_readout_tc(h, *_readout_args(params['readout'][2])))
    return scores[0] + scores[1] + scores[2]
